# baseline (speedup 1.0000x reference)
import functools

import jax
import jax.numpy as jnp
from jax import lax
from jax.experimental import pallas as pl
from jax.experimental.pallas import tpu as pltpu
from jax.experimental.pallas import tpu_sc as plsc

N_ROWS = 64
N_COLS = 8192
KTH = 256
NUM_CORES = 2
NUM_SUBCORES = 16
NW = NUM_CORES * NUM_SUBCORES
ROWS_PER_W = N_ROWS // NW
L = 16
U = 8
UL = U * L
NBINS = 256
TOP_I = -(2 ** 31)


def _sc_kthvalue(x_bits):
    mesh = plsc.VectorSubcoreMesh(
        core_axis_name="c", subcore_axis_name="s",
        num_cores=NUM_CORES, num_subcores=NUM_SUBCORES)

    @functools.partial(
        pl.kernel,
        out_type=(jax.ShapeDtypeStruct((NW, L), jnp.int32),
                  jax.ShapeDtypeStruct((NW, L), jnp.int32)),
        mesh=mesh,
        compiler_params=pltpu.CompilerParams(needs_layout_passes=False),
        scratch_types=[
            pltpu.VMEM((N_COLS,), jnp.int32),
            pltpu.VMEM((N_COLS,), jnp.int32),
            pltpu.VMEM((NBINS,), jnp.int32),
            pltpu.VMEM((NBINS,), jnp.int32),
            pltpu.VMEM((NBINS,), jnp.int32),
            pltpu.VMEM((NBINS,), jnp.int32),
            pltpu.VMEM((L,), jnp.int32),
            pltpu.VMEM((L,), jnp.int32),
            pltpu.VMEM((L,), jnp.int32),
        ],
    )
    def body(x_hbm, vout_hbm, iout_hbm, kbuf0, kbuf1, histA, histB, posaA,
             posaB, bfly, vstage, istage):
        wid = lax.axis_index("s") * NUM_CORES + lax.axis_index("c")
        io = lax.iota(jnp.int32, L)
        perms = tuple(lax.bitwise_xor(io, jnp.int32(1 << p))
                      for p in range(3, -1, -1))
        one = jnp.int32(1)
        zero = jnp.int32(0)
        top = jnp.int32(TOP_I)
        zvec = jnp.zeros((L,), jnp.int32)
        ones_v = jnp.full((L,), 1, jnp.int32)

        kbufs = (kbuf0, kbuf1)
        hists = (histA, histB)
        posas = (posaA, posaB)

        def popc(mask):
            return plsc.all_reduce_population_count(mask)

        def lane_sum(v):
            for p in perms:
                bfly[...] = v
                v = v + plsc.load_gather(bfly, [p])
            return v

        def zero_bins(ref):
            for j in range(NBINS // L):
                ref[pl.ds(j * L, L)] = zvec

        def hist_scan(hist, r):
            run = zero
            less = zvec
            for j in range(NBINS // L):
                h = hist[pl.ds(j * L, L)]
                csg = plsc.cumsum(h) + run
                hist[pl.ds(j * L, L)] = csg
                run = csg[15]
                less = less + popc(csg < r)
            b = less[0]
            bm1 = jnp.maximum(b - 1, zero)
            prev = plsc.load_gather(hist, [jnp.full((L,), bm1, jnp.int32)])
            cbelow = jnp.where(b == 0, zero, prev[0])
            cum_b = plsc.load_gather(hist, [jnp.full((L,), b, jnp.int32)])
            nbin = cum_b[0] - cbelow
            return b, cbelow, nbin

        for row in range(ROWS_PER_W):
            pltpu.sync_copy(x_hbm.at[wid * ROWS_PER_W + row], kbufs[row])

        zero_bins(histA)
        zero_bins(histB)

        @plsc.parallel_loop(0, N_COLS // L, unroll=U)
        def pass_h1(j):
            bs = j * L
            for row in range(ROWS_PER_W):
                krow = kbufs[row]
                b = krow[pl.ds(bs, L)]
                b = jnp.where(b == top, zero, b)
                m = lax.shift_right_arithmetic(b, 31)
                key = lax.bitwise_xor(b, lax.bitwise_or(m, top))
                krow[pl.ds(bs, L)] = key
                bin1 = lax.shift_right_logical(key, 24)
                plsc.addupdate_scatter(hists[row], [bin1], ones_v)

        rs, b1s = [], []
        for row in range(ROWS_PER_W):
            b1, cb1, _ = hist_scan(hists[row], jnp.int32(KTH))
            b1s.append(b1)
            rs.append(jnp.int32(KTH) - cb1)

        zero_bins(histA)
        zero_bins(histB)

        @plsc.parallel_loop(0, N_COLS // L, unroll=U)
        def pass_h2(j):
            bs = j * L
            for row in range(ROWS_PER_W):
                key = kbufs[row][pl.ds(bs, L)]
                m = lax.shift_right_logical(key, 24) == b1s[row]
                binv = lax.bitwise_and(
                    lax.shift_right_logical(key, 16), jnp.int32(0xFF))
                plsc.addupdate_scatter(hists[row], [binv], ones_v, mask=m)

        p16s = []
        for row in range(ROWS_PER_W):
            b2, cb2, _ = hist_scan(hists[row], rs[row])
            rs[row] = rs[row] - cb2
            p16s.append(lax.bitwise_or(
                lax.shift_left(b1s[row], jnp.int32(8)), b2))

        zero_bins(histA)
        zero_bins(histB)
        zero_bins(posaA)
        zero_bins(posaB)

        @plsc.parallel_loop(0, N_COLS // L, unroll=U)
        def pass_h3(j):
            bs = j * L
            for row in range(ROWS_PER_W):
                key = kbufs[row][pl.ds(bs, L)]
                m = lax.shift_right_logical(key, 16) == p16s[row]
                binv = lax.bitwise_and(
                    lax.shift_right_logical(key, 8), jnp.int32(0xFF))
                plsc.addupdate_scatter(hists[row], [binv], ones_v, mask=m)
                plsc.addupdate_scatter(posas[row], [binv], io + bs, mask=m)

        res_v = zvec
        res_i = zvec
        for row in range(ROWS_PER_W):
            krow = kbufs[row]
            hist = hists[row]
            posa = posas[row]
            b3, cb3, n3 = hist_scan(hist, rs[row])
            r = rs[row] - cb3
            p24 = lax.bitwise_or(lax.shift_left(p16s[row], jnp.int32(8)), b3)

            def fast3(_, posa=posa, krow=krow, b3=b3):
                idxv = plsc.load_gather(
                    posa, [jnp.full((L,), b3, jnp.int32)])
                keyv = plsc.load_gather(krow, [idxv])
                return keyv, idxv

            def slow3(_, krow=krow, hist=hist, posa=posa, p24=p24, r=r):
                zero_bins(hist)
                zero_bins(posa)

                @plsc.parallel_loop(0, N_COLS // L, unroll=U)
                def pass_h4(j):
                    bs = j * L
                    key = krow[pl.ds(bs, L)]
                    m = lax.shift_right_logical(key, 8) == p24
                    binv = lax.bitwise_and(key, jnp.int32(0xFF))
                    plsc.addupdate_scatter(hist, [binv], ones_v, mask=m)
                    plsc.addupdate_scatter(posa, [binv], io + bs, mask=m)
                b4, cb4, n4 = hist_scan(hist, r)
                r4 = r - cb4
                v_ans = lax.bitwise_or(lax.shift_left(p24, jnp.int32(8)), b4)

                def fast4(_):
                    return plsc.load_gather(
                        posa, [jnp.full((L,), b4, jnp.int32)])

                def slow4(_):
                    @plsc.parallel_loop(0, N_COLS // L, unroll=U,
                                        carry=(zvec, zvec))
                    def pass_i(j, carry):
                        cnt, pos = carry
                        bs = j * L
                        key = krow[pl.ds(bs, L)]
                        match = key == v_ans
                        mi = jnp.where(match, one, zero)
                        csg = plsc.cumsum(mi) + cnt
                        hit = jnp.logical_and(match, csg == r4)
                        pos = pos + jnp.where(hit, io + bs, zero)
                        cnt = cnt + popc(match)
                        return cnt, pos

                    _, pos_acc = pass_i
                    return lane_sum(pos_acc)

                posv = lax.cond(n4 == 1, fast4, slow4, zero)
                return zvec + v_ans, posv

            key_vec, pos_vec = lax.cond(n3 == 1, fast3, slow3, zero)

            lane = io == row
            res_v = jnp.where(lane, key_vec, res_v)
            res_i = jnp.where(lane, pos_vec, res_i)

        inv = jnp.where(res_v < 0,
                        lax.bitwise_xor(res_v, top),
                        lax.bitwise_xor(res_v, jnp.int32(-1)))
        vstage[...] = inv
        istage[...] = res_i
        pltpu.sync_copy(vstage, vout_hbm.at[wid])
        pltpu.sync_copy(istage, iout_hbm.at[wid])

    return body(x_bits)


def kernel(x):
    xb = lax.bitcast_convert_type(x, jnp.int32)
    vbits, inds = _sc_kthvalue(xb)
    values = lax.bitcast_convert_type(
        vbits[:, :ROWS_PER_W].reshape(N_ROWS), jnp.float32)
    indices = inds[:, :ROWS_PER_W].reshape(N_ROWS)
    return values, indices.astype(jnp.int64)


# speedup vs baseline: 1.0090x; 1.0090x over previous
import functools

import jax
import jax.numpy as jnp
from jax import lax
from jax.experimental import pallas as pl
from jax.experimental.pallas import tpu as pltpu
from jax.experimental.pallas import tpu_sc as plsc

N_ROWS = 64
N_COLS = 8192
KTH = 256
NUM_CORES = 2
NUM_SUBCORES = 16
NW = NUM_CORES * NUM_SUBCORES
ROWS_PER_W = N_ROWS // NW
L = 16
U = 4
UL = U * L
NBINS = 256
TOP_I = -(2 ** 31)


def _sc_kthvalue(x_bits):
    mesh = plsc.VectorSubcoreMesh(
        core_axis_name="c", subcore_axis_name="s",
        num_cores=NUM_CORES, num_subcores=NUM_SUBCORES)

    @functools.partial(
        pl.kernel,
        out_type=(jax.ShapeDtypeStruct((NW, L), jnp.int32),
                  jax.ShapeDtypeStruct((NW, L), jnp.int32)),
        mesh=mesh,
        compiler_params=pltpu.CompilerParams(needs_layout_passes=False),
        scratch_types=[
            pltpu.VMEM((N_COLS,), jnp.int32),
            pltpu.VMEM((N_COLS,), jnp.int32),
            pltpu.VMEM((NBINS,), jnp.int32),
            pltpu.VMEM((NBINS,), jnp.int32),
            pltpu.VMEM((NBINS,), jnp.int32),
            pltpu.VMEM((NBINS,), jnp.int32),
            pltpu.VMEM((L,), jnp.int32),
            pltpu.VMEM((L,), jnp.int32),
            pltpu.VMEM((L,), jnp.int32),
        ],
    )
    def body(x_hbm, vout_hbm, iout_hbm, kbuf0, kbuf1, histA, histB, posaA,
             posaB, bfly, vstage, istage):
        wid = lax.axis_index("s") * NUM_CORES + lax.axis_index("c")
        io = lax.iota(jnp.int32, L)
        perms = tuple(lax.bitwise_xor(io, jnp.int32(1 << p))
                      for p in range(3, -1, -1))
        one = jnp.int32(1)
        zero = jnp.int32(0)
        top = jnp.int32(TOP_I)
        zvec = jnp.zeros((L,), jnp.int32)
        ones_v = jnp.full((L,), 1, jnp.int32)

        kbufs = (kbuf0, kbuf1)
        hists = (histA, histB)
        posas = (posaA, posaB)

        def popc(mask):
            return plsc.all_reduce_population_count(mask)

        def lane_sum(v):
            for p in perms:
                bfly[...] = v
                v = v + plsc.load_gather(bfly, [p])
            return v

        def zero_bins(ref):
            for j in range(NBINS // L):
                ref[pl.ds(j * L, L)] = zvec

        def hist_scan(hist, r):
            run = zero
            less = zvec
            for j in range(NBINS // L):
                h = hist[pl.ds(j * L, L)]
                csg = plsc.cumsum(h) + run
                hist[pl.ds(j * L, L)] = csg
                run = csg[15]
                less = less + popc(csg < r)
            b = less[0]
            bm1 = jnp.maximum(b - 1, zero)
            prev = plsc.load_gather(hist, [jnp.full((L,), bm1, jnp.int32)])
            cbelow = jnp.where(b == 0, zero, prev[0])
            cum_b = plsc.load_gather(hist, [jnp.full((L,), b, jnp.int32)])
            nbin = cum_b[0] - cbelow
            return b, cbelow, nbin

        for row in range(ROWS_PER_W):
            pltpu.sync_copy(x_hbm.at[wid * ROWS_PER_W + row], kbufs[row])

        zero_bins(histA)
        zero_bins(histB)

        @plsc.parallel_loop(0, N_COLS // L, unroll=U)
        def pass_h1(j):
            bs = j * L
            for row in range(ROWS_PER_W):
                krow = kbufs[row]
                b = krow[pl.ds(bs, L)]
                b = jnp.where(b == top, zero, b)
                m = lax.shift_right_arithmetic(b, 31)
                key = lax.bitwise_xor(b, lax.bitwise_or(m, top))
                krow[pl.ds(bs, L)] = key
                bin1 = lax.shift_right_logical(key, 24)
                plsc.addupdate_scatter(hists[row], [bin1], ones_v)

        rs, b1s = [], []
        for row in range(ROWS_PER_W):
            b1, cb1, _ = hist_scan(hists[row], jnp.int32(KTH))
            b1s.append(b1)
            rs.append(jnp.int32(KTH) - cb1)

        zero_bins(histA)
        zero_bins(histB)

        @plsc.parallel_loop(0, N_COLS // L, unroll=U)
        def pass_h2(j):
            bs = j * L
            for row in range(ROWS_PER_W):
                key = kbufs[row][pl.ds(bs, L)]
                m = lax.shift_right_logical(key, 24) == b1s[row]
                binv = lax.bitwise_and(
                    lax.shift_right_logical(key, 16), jnp.int32(0xFF))
                plsc.addupdate_scatter(hists[row], [binv], ones_v, mask=m)

        p16s = []
        for row in range(ROWS_PER_W):
            b2, cb2, _ = hist_scan(hists[row], rs[row])
            rs[row] = rs[row] - cb2
            p16s.append(lax.bitwise_or(
                lax.shift_left(b1s[row], jnp.int32(8)), b2))

        zero_bins(histA)
        zero_bins(histB)
        zero_bins(posaA)
        zero_bins(posaB)

        @plsc.parallel_loop(0, N_COLS // L, unroll=U)
        def pass_h3(j):
            bs = j * L
            for row in range(ROWS_PER_W):
                key = kbufs[row][pl.ds(bs, L)]
                m = lax.shift_right_logical(key, 16) == p16s[row]
                binv = lax.bitwise_and(
                    lax.shift_right_logical(key, 8), jnp.int32(0xFF))
                plsc.addupdate_scatter(hists[row], [binv], ones_v, mask=m)
                plsc.addupdate_scatter(posas[row], [binv], io + bs, mask=m)

        res_v = zvec
        res_i = zvec
        for row in range(ROWS_PER_W):
            krow = kbufs[row]
            hist = hists[row]
            posa = posas[row]
            b3, cb3, n3 = hist_scan(hist, rs[row])
            r = rs[row] - cb3
            p24 = lax.bitwise_or(lax.shift_left(p16s[row], jnp.int32(8)), b3)

            def fast3(_, posa=posa, krow=krow, b3=b3):
                idxv = plsc.load_gather(
                    posa, [jnp.full((L,), b3, jnp.int32)])
                keyv = plsc.load_gather(krow, [idxv])
                return keyv, idxv

            def slow3(_, krow=krow, hist=hist, posa=posa, p24=p24, r=r):
                zero_bins(hist)
                zero_bins(posa)

                @plsc.parallel_loop(0, N_COLS // L, unroll=U)
                def pass_h4(j):
                    bs = j * L
                    key = krow[pl.ds(bs, L)]
                    m = lax.shift_right_logical(key, 8) == p24
                    binv = lax.bitwise_and(key, jnp.int32(0xFF))
                    plsc.addupdate_scatter(hist, [binv], ones_v, mask=m)
                    plsc.addupdate_scatter(posa, [binv], io + bs, mask=m)
                b4, cb4, n4 = hist_scan(hist, r)
                r4 = r - cb4
                v_ans = lax.bitwise_or(lax.shift_left(p24, jnp.int32(8)), b4)

                def fast4(_):
                    return plsc.load_gather(
                        posa, [jnp.full((L,), b4, jnp.int32)])

                def slow4(_):
                    @plsc.parallel_loop(0, N_COLS // L, unroll=U,
                                        carry=(zvec, zvec))
                    def pass_i(j, carry):
                        cnt, pos = carry
                        bs = j * L
                        key = krow[pl.ds(bs, L)]
                        match = key == v_ans
                        mi = jnp.where(match, one, zero)
                        csg = plsc.cumsum(mi) + cnt
                        hit = jnp.logical_and(match, csg == r4)
                        pos = pos + jnp.where(hit, io + bs, zero)
                        cnt = cnt + popc(match)
                        return cnt, pos

                    _, pos_acc = pass_i
                    return lane_sum(pos_acc)

                posv = lax.cond(n4 == 1, fast4, slow4, zero)
                return zvec + v_ans, posv

            key_vec, pos_vec = lax.cond(n3 == 1, fast3, slow3, zero)

            lane = io == row
            res_v = jnp.where(lane, key_vec, res_v)
            res_i = jnp.where(lane, pos_vec, res_i)

        inv = jnp.where(res_v < 0,
                        lax.bitwise_xor(res_v, top),
                        lax.bitwise_xor(res_v, jnp.int32(-1)))
        vstage[...] = inv
        istage[...] = res_i
        pltpu.sync_copy(vstage, vout_hbm.at[wid])
        pltpu.sync_copy(istage, iout_hbm.at[wid])

    return body(x_bits)


def kernel(x):
    xb = lax.bitcast_convert_type(x, jnp.int32)
    vbits, inds = _sc_kthvalue(xb)
    values = lax.bitcast_convert_type(
        vbits[:, :ROWS_PER_W].reshape(N_ROWS), jnp.float32)
    indices = inds[:, :ROWS_PER_W].reshape(N_ROWS)
    return values, indices.astype(jnp.int64)
